# baseline (speedup 1.0000x reference)
import dataclasses

import jax
import jax.numpy as jnp
from jax import lax
from jax.experimental import pallas as pl
from jax.experimental.pallas import tpu as pltpu
from jax.experimental.pallas import tpu_sc as plsc

_LANES = 16
_TILES = 32


def _stats_body(x_ref, fl_ref, ce_ref, w_ref):
    x = x_ref[0]
    L, D = x.shape
    rng = jax.lax.broadcasted_iota(jnp.int32, (1, D), 1).astype(jnp.float32) - D / 2.0
    px = jnp.sum(x, axis=0, keepdims=True)
    px = px / jnp.sum(px)
    mean = jnp.sum(px * rng)
    std = jnp.sqrt(jnp.sum(px * (rng - mean) ** 2))
    new_idx = (rng - mean) / std + D / 2.0
    ii = new_idx.astype(jnp.int32)
    fl_ref[0] = jnp.clip(ii, 0, D - 1)
    ce_ref[0] = jnp.clip(ii + 1, 0, D - 1)
    w_ref[0] = new_idx - jnp.floor(new_idx)


def _stats(distance):
    b, l, d = distance.shape
    i32 = jnp.int32
    fl, ce, w = pl.pallas_call(
        _stats_body,
        grid=(b,),
        in_specs=[pl.BlockSpec((1, l, d), lambda i: (i, 0, 0))],
        out_specs=[
            pl.BlockSpec((1, 1, d), lambda i: (i, 0, 0)),
            pl.BlockSpec((1, 1, d), lambda i: (i, 0, 0)),
            pl.BlockSpec((1, 1, d), lambda i: (i, 0, 0)),
        ],
        out_shape=[
            jax.ShapeDtypeStruct((b, 1, d), i32),
            jax.ShapeDtypeStruct((b, 1, d), i32),
            jax.ShapeDtypeStruct((b, 1, d), jnp.float32),
        ],
    )(distance)
    return fl.reshape(b, d), ce.reshape(b, d), w.reshape(b, d)


def _sc_gather_body(x_hbm, fl_hbm, ce_hbm, w_hbm, o_hbm, rows_in, rows_out,
                    flv, cev, wv):
    b, l, d = x_hbm.shape
    rows_per_tile = l // _TILES
    wid = lax.axis_index("s") * 2 + lax.axis_index("c")
    rbase = wid * rows_per_tile

    @pl.loop(0, b)
    def _batch(bi):
        pltpu.sync_copy(fl_hbm.at[bi], flv)
        pltpu.sync_copy(ce_hbm.at[bi], cev)
        pltpu.sync_copy(w_hbm.at[bi], wv)
        pltpu.sync_copy(x_hbm.at[bi, pl.ds(rbase, rows_per_tile)], rows_in)

        @pl.loop(0, d, step=_LANES)
        def _grp(c):
            f_idx = flv[pl.ds(c, _LANES)]
            c_idx = cev[pl.ds(c, _LANES)]
            wvec = wv[pl.ds(c, _LANES)]

            @pl.loop(0, rows_per_tile)
            def _row(r):
                rvec = jnp.full((_LANES,), 0, jnp.int32) + r
                gf = plsc.load_gather(rows_in, [rvec, f_idx])
                gc = plsc.load_gather(rows_in, [rvec, c_idx])
                rows_out[r, pl.ds(c, _LANES)] = gf + wvec * (gc - gf)

        pltpu.sync_copy(rows_out, o_hbm.at[bi, pl.ds(rbase, rows_per_tile)])


def kernel(distance):
    b, l, d = distance.shape
    fl, ce, w = _stats(distance)
    rows_per_tile = l // _TILES
    mesh = plsc.VectorSubcoreMesh(core_axis_name="c", subcore_axis_name="s")
    cp = pltpu.CompilerParams()
    if "needs_layout_passes" in pltpu.CompilerParams.__dataclass_fields__:
        cp = dataclasses.replace(cp, needs_layout_passes=False)
    sc_gather = pl.kernel(
        _sc_gather_body,
        out_type=jax.ShapeDtypeStruct((b, l, d), jnp.float32),
        mesh=mesh,
        scratch_types=[
            pltpu.VMEM((rows_per_tile, d), jnp.float32),
            pltpu.VMEM((rows_per_tile, d), jnp.float32),
            pltpu.VMEM((d,), jnp.int32),
            pltpu.VMEM((d,), jnp.int32),
            pltpu.VMEM((d,), jnp.float32),
        ],
        compiler_params=cp,
    )
    return sc_gather(distance, fl, ce, w)


# speedup vs baseline: 8.1308x; 8.1308x over previous
import jax
import jax.numpy as jnp
from jax.experimental import pallas as pl


def _body(x_ref, o_ref):
    x = x_ref[0]
    L, D = x.shape
    xb = x.astype(jnp.bfloat16)
    ones = jnp.ones((8, L), jnp.bfloat16)
    px8 = jax.lax.dot(ones, xb, preferred_element_type=jnp.float32)
    px = px8[0:1]
    rng = jax.lax.broadcasted_iota(jnp.int32, (1, D), 1).astype(jnp.float32) - D / 2.0
    px = px / jnp.sum(px)
    mean = jnp.sum(px * rng)
    std = jnp.sqrt(jnp.sum(px * (rng - mean) ** 2))
    new_idx = (rng - mean) / std + D / 2.0
    ii = new_idx.astype(jnp.int32)
    fl = jnp.clip(ii, 0, D - 1)
    ce = jnp.clip(ii + 1, 0, D - 1)
    w = new_idx - jnp.floor(new_idx)
    rows = jax.lax.broadcasted_iota(jnp.int32, (D, D), 0)
    g = jnp.where(rows == fl, 1.0 - w, 0.0) + jnp.where(rows == ce, w, 0.0)
    o_ref[0] = jax.lax.dot(
        xb, g.astype(jnp.bfloat16), preferred_element_type=jnp.float32
    )


def kernel(distance):
    b, l, d = distance.shape
    return pl.pallas_call(
        _body,
        grid=(b,),
        in_specs=[pl.BlockSpec((1, l, d), lambda i: (i, 0, 0))],
        out_specs=pl.BlockSpec((1, l, d), lambda i: (i, 0, 0)),
        out_shape=jax.ShapeDtypeStruct((b, l, d), distance.dtype),
    )(distance)
